# baseline (speedup 1.0000x reference)
import jax
import jax.numpy as jnp
from jax import lax
from jax.experimental import pallas as pl
from jax.experimental.pallas import tpu as pltpu

B = 1
S = 2048
REL_NUM = 16
HEAD = 64
ROWS_PER_BLOCK = 16


def _fanout_body(e_sm, e_row_ref, w_ref, out_ref, slab_ref):
    i = pl.program_id(0)

    @pl.when(i == 0)
    def _build_slabs():
        w = w_ref[...]
        e_row = e_row_ref[...]
        iota_r = lax.broadcasted_iota(jnp.int32, (REL_NUM, S), 0)
        for a in range(4):
            onehot = (iota_r == (e_row + 4 * a)).astype(jnp.float32)
            slab_ref[a] = lax.dot_general(
                w,
                onehot,
                dimension_numbers=(((0,), (0,)), ((), ())),
                preferred_element_type=jnp.float32,
            )

    for k in range(ROWS_PER_BLOCK):
        a_k = e_sm[i * ROWS_PER_BLOCK + k]
        out_ref[k] = slab_ref[a_k]


@jax.jit
def kernel(evidence_type, W):
    e = evidence_type.reshape(S).astype(jnp.int32)
    e_row = e.reshape(1, S)

    grid_spec = pltpu.PrefetchScalarGridSpec(
        num_scalar_prefetch=1,
        grid=(S // ROWS_PER_BLOCK,),
        in_specs=[
            pl.BlockSpec((1, S), lambda i, e_sm: (0, 0)),
            pl.BlockSpec((REL_NUM, HEAD), lambda i, e_sm: (0, 0)),
        ],
        out_specs=pl.BlockSpec(
            (None, ROWS_PER_BLOCK, HEAD, S), lambda i, e_sm: (0, i, 0, 0)
        ),
        scratch_shapes=[pltpu.VMEM((4, HEAD, S), jnp.float32)],
    )

    out = pl.pallas_call(
        _fanout_body,
        grid_spec=grid_spec,
        out_shape=jax.ShapeDtypeStruct((B, S, HEAD, S), jnp.float32),
    )(e, e_row, W)
    return jnp.transpose(out, (0, 1, 3, 2))
